# baseline (speedup 1.0000x reference)
import functools

import jax
import jax.numpy as jnp
from jax import lax
from jax.experimental import pallas as pl
from jax.experimental.pallas import tpu as pltpu
from jax.experimental.pallas import tpu_sc as plsc

D_MODEL = 64
NUM_WORKERS = 32
CHUNK = 1024


def _make_gather(batch):
    b_per_w = batch // NUM_WORKERS
    n_chunks = b_per_w // CHUNK
    mesh = plsc.VectorSubcoreMesh(core_axis_name="c", subcore_axis_name="s")

    @functools.partial(
        pl.kernel,
        mesh=mesh,
        out_type=jax.ShapeDtypeStruct((batch, D_MODEL), jnp.float32),
        compiler_params=pltpu.CompilerParams(use_tc_tiling_on_sc=False),
        scratch_types=[
            pltpu.VMEM((CHUNK,), jnp.int32),
            pltpu.VMEM((CHUNK, D_MODEL), jnp.float32),
            pltpu.SemaphoreType.DMA,
        ],
    )
    def gather(table_hbm, idx_hbm, out_hbm, idx_v, rows_v, sem):
        wid = lax.axis_index("s") * 2 + lax.axis_index("c")
        base = wid * b_per_w

        def body(i, carry):
            off = base + i * CHUNK
            pltpu.sync_copy(idx_hbm.at[pl.ds(off, CHUNK)], idx_v)
            pltpu.async_copy(table_hbm.at[idx_v], rows_v, sem).wait()
            pltpu.sync_copy(rows_v, out_hbm.at[pl.ds(off, CHUNK)])
            return carry

        lax.fori_loop(0, n_chunks, body, 0)

    return gather


@jax.jit
def kernel(x, embedding_matrix):
    batch = x.shape[0] * x.shape[1]
    idx = x.reshape(batch).astype(jnp.int32)
    out = _make_gather(batch)(embedding_matrix, idx)
    return out.reshape(x.shape[0], x.shape[1], D_MODEL)


# speedup vs baseline: 1.6643x; 1.6643x over previous
import functools

import jax
import jax.numpy as jnp
from jax import lax
from jax.experimental import pallas as pl
from jax.experimental.pallas import tpu as pltpu
from jax.experimental.pallas import tpu_sc as plsc

VOCAB = 1000000
D_MODEL = 64
NUM_WORKERS = 32
CHUNK = 1024
TBLK = 7936
SPLIT = 499968
H_ROWS = 500032


def _transpose_body(t1_ref, t2_ref, o_ref):
    o_ref[...] = jnp.concatenate([t1_ref[...].T, t2_ref[...].T], axis=1)


_transpose = pl.pallas_call(
    _transpose_body,
    grid=(pl.cdiv(H_ROWS, TBLK),),
    in_specs=[
        pl.BlockSpec((D_MODEL, TBLK), lambda i: (0, i)),
        pl.BlockSpec((D_MODEL, TBLK), lambda i: (0, i + SPLIT // TBLK)),
    ],
    out_specs=pl.BlockSpec((TBLK, 128), lambda i: (i, 0)),
    out_shape=jax.ShapeDtypeStruct((H_ROWS, 128), jnp.float32),
)


def _make_gather(batch):
    b_per_w = batch // NUM_WORKERS
    n_chunks = b_per_w // CHUNK
    mesh = plsc.VectorSubcoreMesh(core_axis_name="c", subcore_axis_name="s")

    @functools.partial(
        pl.kernel,
        mesh=mesh,
        out_type=jax.ShapeDtypeStruct((batch, D_MODEL), jnp.float32),
        compiler_params=pltpu.CompilerParams(use_tc_tiling_on_sc=False),
        scratch_types=[
            pltpu.VMEM((CHUNK,), jnp.int32),
            pltpu.VMEM((CHUNK,), jnp.int32),
            pltpu.VMEM((CHUNK, D_MODEL), jnp.float32),
            pltpu.SemaphoreType.DMA,
        ],
    )
    def gather(table_hbm, idx_hbm, out_hbm, idx_v, idx2_v, rows_v, sem):
        wid = lax.axis_index("s") * 2 + lax.axis_index("c")
        base = wid * b_per_w

        def remap(g, carry):
            v = idx_v[pl.ds(g * 16, 16)]
            u = v + v
            idx2_v[pl.ds(g * 16, 16)] = jnp.where(
                v < SPLIT, u, u - (2 * SPLIT - 1)
            )
            return carry

        def body(i, carry):
            off = base + i * CHUNK
            pltpu.sync_copy(idx_hbm.at[pl.ds(off, CHUNK)], idx_v)
            lax.fori_loop(0, CHUNK // 16, remap, 0)
            pltpu.async_copy(table_hbm.at[idx2_v], rows_v, sem).wait()
            pltpu.sync_copy(rows_v, out_hbm.at[pl.ds(off, CHUNK)])
            return carry

        lax.fori_loop(0, n_chunks, body, 0)

    return gather


@jax.jit
def kernel(x, embedding_matrix):
    batch = x.shape[0] * x.shape[1]
    idx = x.reshape(batch).astype(jnp.int32)
    t_view = embedding_matrix.T
    table_h = _transpose(t_view, t_view)
    table_rm = table_h.reshape(2 * H_ROWS, D_MODEL)
    out = _make_gather(batch)(table_rm, idx)
    return out.reshape(x.shape[0], x.shape[1], D_MODEL)
